# baseline (speedup 1.0000x reference)
import functools

import jax
import jax.numpy as jnp
from jax import lax
from jax.experimental import pallas as pl
from jax.experimental.pallas import tpu as pltpu
from jax.experimental.pallas import tpu_sc as plsc

N_NODES = 10000
D_IN = 500
N_PAD = 10240
BLK_N = 640
GRID_N = N_PAD // BLK_N
E_REAL = 160000 + N_NODES
E_PAD = 172032
CE = 2688
GPC = CE // 16
PAD_IDX = N_NODES

f32 = jnp.float32


def _tc1_body(xT_ref, w1T_ref, asT_ref, adT_ref, hT_ref, aso_ref, ado_ref):
    hT = jnp.dot(w1T_ref[...], xT_ref[...], preferred_element_type=f32)
    hT_ref[...] = hT
    aso_ref[...] = jnp.dot(asT_ref[...], hT, preferred_element_type=f32)
    ado_ref[...] = jnp.dot(adT_ref[...], hT, preferred_element_type=f32)


def _tc1(xT, W1T, A_sT, A_dT):
    return pl.pallas_call(
        _tc1_body,
        grid=(GRID_N,),
        in_specs=[
            pl.BlockSpec((D_IN, BLK_N), lambda i: (0, i)),
            pl.BlockSpec((64, D_IN), lambda i: (0, 0)),
            pl.BlockSpec((8, 64), lambda i: (0, 0)),
            pl.BlockSpec((8, 64), lambda i: (0, 0)),
        ],
        out_specs=[
            pl.BlockSpec((64, BLK_N), lambda i: (0, i)),
            pl.BlockSpec((8, BLK_N), lambda i: (0, i)),
            pl.BlockSpec((8, BLK_N), lambda i: (0, i)),
        ],
        out_shape=[
            jax.ShapeDtypeStruct((64, N_PAD), f32),
            jax.ShapeDtypeStruct((8, N_PAD), f32),
            jax.ShapeDtypeStruct((8, N_PAD), f32),
        ],
    )(xT, W1T, A_sT, A_dT)


def _tc2_body(dp_ref, numa_ref, numb_ref, q_ref, r8_ref, b1_ref, w2T_ref,
              vs_ref, vd_ref, p78_ref, h2p_ref, as2_ref, ad2_ref):
    den8 = jnp.dot(q_ref[...], dp_ref[...], preferred_element_type=f32)
    denr = jnp.dot(r8_ref[...], den8, preferred_element_type=f32)
    out1T = (numa_ref[...] + numb_ref[...]) / (denr + 1e-16) + b1_ref[...]
    h2T = jnp.dot(w2T_ref[...], out1T, preferred_element_type=f32)
    h2p_ref[...] = jnp.dot(p78_ref[...], h2T, preferred_element_type=f32)
    as2_ref[...] = jnp.dot(vs_ref[...], h2T, preferred_element_type=f32)
    ad2_ref[...] = jnp.dot(vd_ref[...], h2T, preferred_element_type=f32)


def _tc2(denp1, numa, numb, Q, R8, b1c, W2T, vs2, vd2, P78):
    return pl.pallas_call(
        _tc2_body,
        grid=(GRID_N,),
        in_specs=[
            pl.BlockSpec((32, BLK_N), lambda i: (0, i)),
            pl.BlockSpec((64, BLK_N), lambda i: (0, i)),
            pl.BlockSpec((64, BLK_N), lambda i: (0, i)),
            pl.BlockSpec((8, 32), lambda i: (0, 0)),
            pl.BlockSpec((64, 8), lambda i: (0, 0)),
            pl.BlockSpec((64, 1), lambda i: (0, 0)),
            pl.BlockSpec((7, 64), lambda i: (0, 0)),
            pl.BlockSpec((1, 7), lambda i: (0, 0)),
            pl.BlockSpec((1, 7), lambda i: (0, 0)),
            pl.BlockSpec((8, 7), lambda i: (0, 0)),
        ],
        out_specs=[
            pl.BlockSpec((8, BLK_N), lambda i: (0, i)),
            pl.BlockSpec((1, BLK_N), lambda i: (0, i)),
            pl.BlockSpec((1, BLK_N), lambda i: (0, i)),
        ],
        out_shape=[
            jax.ShapeDtypeStruct((8, N_PAD), f32),
            jax.ShapeDtypeStruct((1, N_PAD), f32),
            jax.ShapeDtypeStruct((1, N_PAD), f32),
        ],
    )(denp1, numa, numb, Q, R8, b1c, W2T, vs2, vd2, P78)


def _tc3_body(dp_ref, np_ref, ones_ref, b2_ref, out_ref):
    den = jnp.dot(ones_ref[...], dp_ref[...], preferred_element_type=f32)
    nb = np_ref[0]
    for _qq in range(1, 16):
        nb = nb + np_ref[_qq]
    o = nb[0:7, :] / (den + 1e-16) + b2_ref[...]
    m = jnp.max(o, axis=0, keepdims=True)
    e = jnp.exp(o - m)
    out_ref[...] = (o - m) - jnp.log(jnp.sum(e, axis=0, keepdims=True))


def _tc3(denp2, nump2, ones32, b2c):
    return pl.pallas_call(
        _tc3_body,
        grid=(GRID_N,),
        in_specs=[
            pl.BlockSpec((32, BLK_N), lambda i: (0, i)),
            pl.BlockSpec((16, 8, BLK_N), lambda i: (0, 0, i)),
            pl.BlockSpec((1, 32), lambda i: (0, 0)),
            pl.BlockSpec((7, 1), lambda i: (0, 0)),
        ],
        out_specs=pl.BlockSpec((7, BLK_N), lambda i: (0, i)),
        out_shape=jax.ShapeDtypeStruct((7, N_PAD), f32),
    )(denp2, nump2, ones32, b2c)


def _sc_weights(nh):
    nq = 32 // nh
    eq = E_PAD // nq
    nchunk = eq // CE
    mesh = plsc.VectorSubcoreMesh(core_axis_name="c", subcore_axis_name="s",
                                  num_cores=2)

    @functools.partial(
        pl.kernel, mesh=mesh,
        compiler_params=pltpu.CompilerParams(needs_layout_passes=False),
        out_type=[
            jax.ShapeDtypeStruct((nh, E_PAD), f32),
            jax.ShapeDtypeStruct((32, N_PAD), f32),
        ],
        scratch_types=[
            pltpu.VMEM((CE,), jnp.int32),
            pltpu.VMEM((CE,), jnp.int32),
            pltpu.VMEM((CE,), f32),
            pltpu.VMEM((N_PAD,), f32),
            pltpu.VMEM((N_PAD,), f32),
            pltpu.VMEM((N_PAD,), f32),
        ],
    )
    def k(srci, dsti, asT, adT, exo, denp, sidx, didx, exb, asb, adb, den):
        t = lax.axis_index("c") * 16 + lax.axis_index("s")
        hd = t % nh
        q = t // nh

        pltpu.sync_copy(asT.at[hd], asb)
        pltpu.sync_copy(adT.at[hd], adb)

        z16 = jnp.zeros((16,), f32)
        lane = lax.iota(jnp.int32, 16)
        def zrow(i, cc):
            plsc.store_scatter(den, [lane + i * 16], z16)
            return cc
        lax.fori_loop(0, N_PAD // 16, zrow, 0)

        def chunk(ch, cc):
            base = q * eq + ch * CE
            pltpu.sync_copy(srci.at[pl.ds(base, CE)], sidx)
            pltpu.sync_copy(dsti.at[pl.ds(base, CE)], didx)

            def grp(g, gg):
                o = g * 16
                s16 = sidx[pl.ds(o, 16)]
                d16 = didx[pl.ds(o, 16)]
                av = plsc.load_gather(asb, [s16])
                bv = plsc.load_gather(adb, [d16])
                al = av + bv
                lr = jnp.where(al > 0, al, al * 0.2)
                ex = jnp.exp(lr)
                exb[pl.ds(o, 16)] = ex
                plsc.addupdate_scatter(den, [d16], ex)
                return gg
            lax.fori_loop(0, GPC, grp, 0, unroll=16)
            pltpu.sync_copy(exb, exo.at[hd].at[pl.ds(base, CE)])
            return cc
        lax.fori_loop(0, nchunk, chunk, 0)

        pltpu.sync_copy(den, denp.at[t])

    return k


def _sc_numerator(n_rows, rpt):
    ntc = n_rows // rpt
    nq = 32 // ntc
    eq = E_PAD // nq
    nchunk = eq // CE
    mesh = plsc.VectorSubcoreMesh(core_axis_name="c", subcore_axis_name="s",
                                  num_cores=2)

    def make(nh_ex):
        rows_per_head = n_rows // nh_ex

        @functools.partial(
            pl.kernel, mesh=mesh,
            compiler_params=pltpu.CompilerParams(needs_layout_passes=False),
            out_type=jax.ShapeDtypeStruct((nq, n_rows, N_PAD), f32),
            scratch_types=[
                pltpu.VMEM((CE,), jnp.int32),
                pltpu.VMEM((CE,), jnp.int32),
                pltpu.VMEM((CE,), f32),
                pltpu.VMEM((rpt, N_PAD), f32),
                pltpu.VMEM((rpt, N_PAD), f32),
            ],
        )
        def k(srci, dsti, htab, exo, numo, sidx, didx, exb, hbuf, acc):
            t = lax.axis_index("c") * 16 + lax.axis_index("s")
            c0 = (t % ntc) * rpt
            q = t // ntc
            hd = c0 // rows_per_head

            pltpu.sync_copy(htab.at[pl.ds(c0, rpt)], hbuf)

            z16 = jnp.zeros((16,), f32)
            lane = lax.iota(jnp.int32, 16)
            jfs = [jnp.full((16,), j, jnp.int32) for j in range(rpt)]
            def zrow(i, cc):
                for j in range(rpt):
                    plsc.store_scatter(acc, [jfs[j], lane + i * 16], z16)
                return cc
            lax.fori_loop(0, N_PAD // 16, zrow, 0)

            def chunk(ch, cc):
                base = q * eq + ch * CE
                pltpu.sync_copy(srci.at[pl.ds(base, CE)], sidx)
                pltpu.sync_copy(dsti.at[pl.ds(base, CE)], didx)
                pltpu.sync_copy(exo.at[hd].at[pl.ds(base, CE)], exb)

                def grp(g, gg):
                    o = g * 16
                    s16 = sidx[pl.ds(o, 16)]
                    d16 = didx[pl.ds(o, 16)]
                    ex = exb[pl.ds(o, 16)]
                    for j in range(rpt):
                        hv = plsc.load_gather(hbuf, [jfs[j], s16])
                        plsc.addupdate_scatter(acc, [jfs[j], d16], hv * ex)
                    return gg
                lax.fori_loop(0, GPC, grp, 0, unroll=8)
                return cc
            lax.fori_loop(0, nchunk, chunk, 0)

            pltpu.sync_copy(acc, numo.at[q].at[pl.ds(c0, rpt)])

        return k
    return make


_sc_w1 = _sc_weights(8)
_sc_w2 = _sc_weights(1)
_sc_n1 = _sc_numerator(64, 4)(8)
_sc_n2 = _sc_numerator(8, 4)(1)


def kernel(x, edge_index, W1, att_src1, att_dst1, bias1, W2, att_src2,
           att_dst2, bias2):
    loop = jnp.arange(N_NODES, dtype=jnp.int32)
    pad = jnp.full((E_PAD - E_REAL,), PAD_IDX, jnp.int32)
    src = jnp.concatenate([edge_index[0].astype(jnp.int32), loop, pad])
    dst = jnp.concatenate([edge_index[1].astype(jnp.int32), loop, pad])
    xT = jnp.zeros((D_IN, N_PAD), f32).at[:, :N_NODES].set(x.astype(f32).T)

    eye8 = jnp.eye(8, dtype=f32)
    A_sT = (eye8[:, :, None] * att_src1[0].astype(f32)[:, None, :]).reshape(8, 64)
    A_dT = (eye8[:, :, None] * att_dst1[0].astype(f32)[:, None, :]).reshape(8, 64)
    Q = jnp.concatenate([eye8, eye8, eye8, eye8], axis=1)
    R8 = jnp.repeat(eye8, 8, axis=0)
    P78 = jnp.eye(8, 7, dtype=f32)
    ones32 = jnp.ones((1, 32), f32)
    vs2 = att_src2[0].astype(f32)
    vd2 = att_dst2[0].astype(f32)
    b1c = bias1.astype(f32).reshape(64, 1)
    b2c = bias2.astype(f32).reshape(7, 1)

    hT, asT, adT = _tc1(xT, W1.astype(f32).T, A_sT, A_dT)
    ex1, denp1 = _sc_w1(src, dst, asT, adT)
    num1p = _sc_n1(src, dst, hT, ex1)
    h2pT, as2T, ad2T = _tc2(denp1, num1p[0], num1p[1], Q, R8, b1c, W2.astype(f32).T,
                            vs2, vd2, P78)
    ex2, denp2 = _sc_w2(src, dst, as2T, ad2T)
    nump2 = _sc_n2(src, dst, h2pT, ex2)
    outT = _tc3(denp2, nump2, ones32, b2c)
    return outT.T[:N_NODES]
